# baseline (speedup 1.0000x reference)
import functools

import jax
import jax.numpy as jnp
from jax import lax
from jax.experimental import pallas as pl
from jax.experimental.pallas import tpu as pltpu
from jax.experimental.pallas import tpu_sc as plsc

_N, _D, _K = 16384, 64, 4096
_BX = 256

_NC, _NS = 2, 16
_NW = _NC * _NS
_RPW = _N // _NW
_CH = 128
_NCH = _RPW // _CH


_WINDOWS = ((0, 1408), (1408, 2816), (2816, 4096))


_L = 128


def _argmin_body(x_ref, x2_ref, c_ref, c2_ref, idx_ref):
    xb = (x_ref[...] * 2.0).astype(jnp.bfloat16)
    cb = c_ref[...].astype(jnp.bfloat16)
    m2 = lax.dot_general(xb, cb, (((1,), (1,)), ((), ())),
                         preferred_element_type=jnp.float32)
    x2 = x2_ref[...][:, None]
    c2 = c2_ref[...]
    lane = lax.broadcasted_iota(jnp.int32, (_BX, _L), 1)
    acc = None
    for (w0, w1) in _WINDOWS:
        vacc = None
        for k in range(w0 // _L, w1 // _L):
            dk = (x2 - m2[:, k * _L:(k + 1) * _L]) + c2[k * _L:(k + 1) * _L][None, :]
            if vacc is None:
                vacc = dk
                kidx = jnp.full((_BX, _L), k, jnp.int32)
            else:
                cmp = dk < vacc
                vacc = jnp.where(cmp, dk, vacc)
                kidx = jnp.where(cmp, jnp.int32(k), kidx)
        vw = jnp.min(vacc, axis=-1)
        jc = jnp.where(vacc == vw[:, None], kidx * _L + lane, jnp.int32(_K))
        kw = jnp.min(jc, axis=-1)
        vw_r = vw.astype(jnp.bfloat16).astype(jnp.float32)
        if acc is None:
            acc, ai = vw_r, kw
        else:
            win = vw < acc
            acc = jnp.where(win, vw_r, acc)
            ai = jnp.where(win, kw, ai)
    idx_ref[...] = ai


def _compute_idx(x, x2, codebook, c2):
    return pl.pallas_call(
        _argmin_body,
        grid=(_N // _BX,),
        in_specs=[
            pl.BlockSpec((_BX, _D), lambda i: (i, 0)),
            pl.BlockSpec((_BX,), lambda i: (i,)),
            pl.BlockSpec((_K, _D), lambda i: (0, 0)),
            pl.BlockSpec((_K,), lambda i: (0,)),
        ],
        out_specs=pl.BlockSpec((_BX,), lambda i: (i,)),
        out_shape=jax.ShapeDtypeStruct((_N,), jnp.int32),
        compiler_params=pltpu.CompilerParams(
            dimension_semantics=("parallel",)),
    )(x, x2, codebook, c2)


def _gather_rows(codebook, idx):
    mesh = plsc.VectorSubcoreMesh(core_axis_name="c", subcore_axis_name="s")

    @functools.partial(
        pl.kernel, mesh=mesh,
        out_type=jax.ShapeDtypeStruct((_N, _D), jnp.float32),
        scratch_types=[
            pltpu.VMEM((_RPW,), jnp.int32),
            pltpu.VMEM((_RPW, _D), jnp.float32),
            pltpu.SemaphoreType.DMA,
        ],
        compiler_params=pltpu.CompilerParams(use_tc_tiling_on_sc=False),
    )
    def gather_k(table_hbm, idx_hbm, out_hbm, idx_v, rows_v, sem):
        wid = lax.axis_index("s") * _NC + lax.axis_index("c")
        base = wid * _RPW
        pltpu.sync_copy(idx_hbm.at[pl.ds(base, _RPW)], idx_v)
        copies = [
            pltpu.async_copy(table_hbm.at[idx_v.at[pl.ds(j * _CH, _CH)]],
                             rows_v.at[pl.ds(j * _CH, _CH)], sem)
            for j in range(_NCH)
        ]
        for cp in copies:
            cp.wait()
        pltpu.sync_copy(rows_v, out_hbm.at[pl.ds(base, _RPW)])

    return gather_k(codebook, idx)


def kernel(x, codebook):
    x2 = jnp.sum(x * x, axis=-1, keepdims=True)[:, 0]
    c2 = jnp.sum(codebook * codebook, axis=-1)
    idx = _compute_idx(x, x2, codebook, c2)
    quantized = _gather_rows(codebook, idx)
    return quantized, idx


# speedup vs baseline: 1.0616x; 1.0616x over previous
import functools

import jax
import jax.numpy as jnp
from jax import lax
from jax.experimental import pallas as pl
from jax.experimental.pallas import tpu as pltpu
from jax.experimental.pallas import tpu_sc as plsc

_N, _D, _K = 16384, 64, 4096
_BX = 256

_NC, _NS = 2, 16
_NW = _NC * _NS
_RPW = _N // _NW
_CH = 128
_NCH = _RPW // _CH


_WINDOWS = ((0, 1408), (1408, 2816), (2816, 4096))


_L = 128


def _argmin_body(x_ref, x2_ref, c_ref, c2_ref, idx_ref, cb_ref):
    @pl.when(pl.program_id(0) == 0)
    def _():
        cb_ref[...] = c_ref[...].astype(jnp.bfloat16)

    xb = (x_ref[...] * 2.0).astype(jnp.bfloat16)
    m2 = lax.dot_general(xb, cb_ref[...], (((1,), (1,)), ((), ())),
                         preferred_element_type=jnp.float32)
    x2 = x2_ref[...][:, None]
    c2 = c2_ref[...]
    lane = lax.broadcasted_iota(jnp.int32, (_BX, _L), 1).astype(jnp.float32)
    acc = None
    for (w0, w1) in _WINDOWS:
        vacc = None
        for k in range(w0 // _L, w1 // _L):
            dk = (x2 - m2[:, k * _L:(k + 1) * _L]) + c2[k * _L:(k + 1) * _L][None, :]
            if vacc is None:
                vacc = dk
                kidx = jnp.full((_BX, _L), float(k * _L), jnp.float32)
            else:
                cmp = dk < vacc
                vacc = jnp.where(cmp, dk, vacc)
                kidx = jnp.where(cmp, jnp.float32(k * _L), kidx)
        vw = jnp.min(vacc, axis=-1)
        jc = jnp.where(vacc == vw[:, None], kidx + lane, jnp.float32(_K))
        kw = jnp.min(jc, axis=-1)
        vw_r = vw.astype(jnp.bfloat16).astype(jnp.float32)
        if acc is None:
            acc, ai = vw_r, kw
        else:
            win = vw < acc
            acc = jnp.where(win, vw_r, acc)
            ai = jnp.where(win, kw, ai)
    idx_ref[...] = ai.astype(jnp.int32)


def _compute_idx(x, x2, codebook, c2):
    return pl.pallas_call(
        _argmin_body,
        grid=(_N // _BX,),
        in_specs=[
            pl.BlockSpec((_BX, _D), lambda i: (i, 0)),
            pl.BlockSpec((_BX,), lambda i: (i,)),
            pl.BlockSpec((_K, _D), lambda i: (0, 0)),
            pl.BlockSpec((_K,), lambda i: (0,)),
        ],
        out_specs=pl.BlockSpec((_BX,), lambda i: (i,)),
        out_shape=jax.ShapeDtypeStruct((_N,), jnp.int32),
        scratch_shapes=[pltpu.VMEM((_K, _D), jnp.bfloat16)],
    )(x, x2, codebook, c2)


def _gather_rows(codebook, idx):
    mesh = plsc.VectorSubcoreMesh(core_axis_name="c", subcore_axis_name="s")

    @functools.partial(
        pl.kernel, mesh=mesh,
        out_type=jax.ShapeDtypeStruct((_N, _D), jnp.float32),
        scratch_types=[
            pltpu.VMEM((_RPW,), jnp.int32),
            pltpu.VMEM((_RPW, _D), jnp.float32),
            pltpu.SemaphoreType.DMA,
        ],
        compiler_params=pltpu.CompilerParams(use_tc_tiling_on_sc=False),
    )
    def gather_k(table_hbm, idx_hbm, out_hbm, idx_v, rows_v, sem):
        wid = lax.axis_index("s") * _NC + lax.axis_index("c")
        base = wid * _RPW
        pltpu.sync_copy(idx_hbm.at[pl.ds(base, _RPW)], idx_v)
        copies = [
            pltpu.async_copy(table_hbm.at[idx_v.at[pl.ds(j * _CH, _CH)]],
                             rows_v.at[pl.ds(j * _CH, _CH)], sem)
            for j in range(_NCH)
        ]
        for cp in copies:
            cp.wait()
        pltpu.sync_copy(rows_v, out_hbm.at[pl.ds(base, _RPW)])

    return gather_k(codebook, idx)


def kernel(x, codebook):
    x2 = jnp.sum(x * x, axis=-1, keepdims=True)[:, 0]
    c2 = jnp.sum(codebook * codebook, axis=-1)
    idx = _compute_idx(x, x2, codebook, c2)
    quantized = _gather_rows(codebook, idx)
    return quantized, idx


# speedup vs baseline: 1.1530x; 1.0861x over previous
import functools

import jax
import jax.numpy as jnp
from jax import lax
from jax.experimental import pallas as pl
from jax.experimental.pallas import tpu as pltpu
from jax.experimental.pallas import tpu_sc as plsc

_N, _D, _K = 16384, 64, 4096
_BX = 512
_RT = 64

_NC, _NS = 2, 16
_NW = _NC * _NS
_RPW = _N // _NW
_CH = 128
_NCH = _RPW // _CH


_WINDOWS = ((0, 1408), (1408, 2816), (2816, 4096))


_L = 128


def _argmin_body(x_ref, x2_ref, c_ref, c2_ref, idx_ref, cb_ref):
    @pl.when(pl.program_id(0) == 0)
    def _():
        cb_ref[...] = c_ref[...].astype(jnp.bfloat16)

    xb = (x_ref[...] * 2.0).astype(jnp.bfloat16)
    m2 = lax.dot_general(xb, cb_ref[...], (((1,), (1,)), ((), ())),
                         preferred_element_type=jnp.float32)
    x2a = x2_ref[...][:, None]
    c2 = c2_ref[...]
    lane = lax.broadcasted_iota(jnp.int32, (_RT, _L), 1).astype(jnp.float32)
    for r in range(0, _BX, _RT):
        x2 = x2a[r:r + _RT]
        acc = None
        for (w0, w1) in _WINDOWS:
            vacc = None
            for k in range(w0 // _L, w1 // _L):
                dk = (x2 - m2[r:r + _RT, k * _L:(k + 1) * _L]) \
                    + c2[k * _L:(k + 1) * _L][None, :]
                if vacc is None:
                    vacc = dk
                    kidx = jnp.full((_RT, _L), float(k * _L), jnp.float32)
                else:
                    cmp = dk < vacc
                    vacc = jnp.where(cmp, dk, vacc)
                    kidx = jnp.where(cmp, jnp.float32(k * _L), kidx)
            vw = jnp.min(vacc, axis=-1)
            jc = jnp.where(vacc == vw[:, None], kidx + lane, jnp.float32(_K))
            kw = jnp.min(jc, axis=-1)
            vw_r = vw.astype(jnp.bfloat16).astype(jnp.float32)
            if acc is None:
                acc, ai = vw_r, kw
            else:
                win = vw < acc
                acc = jnp.where(win, vw_r, acc)
                ai = jnp.where(win, kw, ai)
        idx_ref[r:r + _RT] = ai.astype(jnp.int32)


def _compute_idx(x, x2, codebook, c2):
    return pl.pallas_call(
        _argmin_body,
        grid=(_N // _BX,),
        in_specs=[
            pl.BlockSpec((_BX, _D), lambda i: (i, 0)),
            pl.BlockSpec((_BX,), lambda i: (i,)),
            pl.BlockSpec((_K, _D), lambda i: (0, 0)),
            pl.BlockSpec((_K,), lambda i: (0,)),
        ],
        out_specs=pl.BlockSpec((_BX,), lambda i: (i,)),
        out_shape=jax.ShapeDtypeStruct((_N,), jnp.int32),
        scratch_shapes=[pltpu.VMEM((_K, _D), jnp.bfloat16)],
    )(x, x2, codebook, c2)


def _gather_rows(codebook, idx):
    mesh = plsc.VectorSubcoreMesh(core_axis_name="c", subcore_axis_name="s")

    @functools.partial(
        pl.kernel, mesh=mesh,
        out_type=jax.ShapeDtypeStruct((_N, _D), jnp.float32),
        scratch_types=[
            pltpu.VMEM((_RPW,), jnp.int32),
            pltpu.VMEM((_RPW, _D), jnp.float32),
            pltpu.SemaphoreType.DMA,
        ],
        compiler_params=pltpu.CompilerParams(use_tc_tiling_on_sc=False),
    )
    def gather_k(table_hbm, idx_hbm, out_hbm, idx_v, rows_v, sem):
        wid = lax.axis_index("s") * _NC + lax.axis_index("c")
        base = wid * _RPW
        pltpu.sync_copy(idx_hbm.at[pl.ds(base, _RPW)], idx_v)
        copies = [
            pltpu.async_copy(table_hbm.at[idx_v.at[pl.ds(j * _CH, _CH)]],
                             rows_v.at[pl.ds(j * _CH, _CH)], sem)
            for j in range(_NCH)
        ]
        for cp in copies:
            cp.wait()
        pltpu.sync_copy(rows_v, out_hbm.at[pl.ds(base, _RPW)])

    return gather_k(codebook, idx)


def kernel(x, codebook):
    x2 = jnp.sum(x * x, axis=-1, keepdims=True)[:, 0]
    c2 = jnp.sum(codebook * codebook, axis=-1)
    idx = _compute_idx(x, x2, codebook, c2)
    quantized = _gather_rows(codebook, idx)
    return quantized, idx
